# Initial kernel scaffold; baseline (speedup 1.0000x reference)
import jax
import jax.numpy as jnp
from jax.experimental import pallas as pl


def kernel(message, landmarks, mask, emb_table, W, b):
    raise NotImplementedError("write your pallas kernel here")


# speedup vs baseline: 20.0020x; 20.0020x over previous
import functools

import jax
import jax.numpy as jnp
from jax import lax
from jax.experimental import pallas as pl
from jax.experimental.pallas import tpu as pltpu
from jax.experimental.pallas import tpu_sc as plsc

_LANES = 16
_NW = 32


def _sc_gather_pool(emb_table, lm2d, L, T):
    N, IDX = lm2d.shape
    E = emb_table.shape[1]
    SPW = N // _NW
    n_full, rem = divmod(IDX, 128)
    chunks = [(c * 128, 128) for c in range(n_full)]
    if rem:
        chunks.append((n_full * 128, rem))
    mesh = plsc.VectorSubcoreMesh(core_axis_name="c", subcore_axis_name="s")

    @functools.partial(
        pl.kernel,
        out_type=jax.ShapeDtypeStruct((N, L * E), jnp.float32),
        mesh=mesh,
        scratch_types=[
            pltpu.VMEM((IDX,), jnp.int32),
            pltpu.VMEM((IDX, E), jnp.float32),
            pltpu.VMEM((SPW, L * E), jnp.float32),
            pltpu.SemaphoreType.DMA,
        ],
        compiler_params=pltpu.CompilerParams(use_tc_tiling_on_sc=False),
    )
    def k(table_hbm, lm_hbm, out_hbm, idx_v, rows_v, out_v, sem):
        wid = lax.axis_index("s") * 2 + lax.axis_index("c")
        base = wid * SPW

        def sample_body(i, carry):
            pltpu.sync_copy(lm_hbm.at[base + i], idx_v)
            cps = [
                pltpu.async_copy(
                    table_hbm.at[idx_v.at[pl.ds(off, sz)]],
                    rows_v.at[pl.ds(off, sz)],
                    sem,
                )
                for off, sz in chunks
            ]
            for cp in cps:
                cp.wait()

            def l_body(l, carry_l):
                def t_body(t, accs):
                    a0, a1 = accs
                    r = l * T + t
                    return (a0 + rows_v[r, pl.ds(0, _LANES)],
                            a1 + rows_v[r, pl.ds(_LANES, _LANES)])

                z = jnp.zeros((_LANES,), jnp.float32)
                a0, a1 = lax.fori_loop(0, T, t_body, (z, z))
                out_v[i, pl.ds(l * E, _LANES)] = a0
                out_v[i, pl.ds(l * E + _LANES, _LANES)] = a1
                return carry_l

            return lax.fori_loop(0, L, l_body, carry)

        lax.fori_loop(0, SPW, sample_body, 0)
        pltpu.sync_copy(out_v, out_hbm.at[pl.ds(base, SPW)])

    return k(emb_table, lm2d)


def _tc_score(message, W, b, mask, pooled3d):
    B, V = message.shape
    E = W.shape[0]
    L = mask.shape[1]
    BB = 256

    def body(msg_ref, w_ref, b_ref, mask_ref, pooled_ref, out_ref):
        m = lax.dot_general(
            msg_ref[...], w_ref[...], (((1,), (1,)), ((), ())),
            preferred_element_type=jnp.float32)
        m = jnp.maximum(m + b_ref[...], 0.0)
        p = jnp.maximum(pooled_ref[...], 0.0)
        s = jnp.sum(p * m[:, None, :], axis=2)
        s = s + (1.0 - mask_ref[...]) * (-1e36)
        mx = jnp.max(s, axis=1, keepdims=True)
        e = jnp.exp(s - mx)
        out_ref[...] = e / jnp.sum(e, axis=1, keepdims=True)

    return pl.pallas_call(
        body,
        grid=(B // BB,),
        in_specs=[
            pl.BlockSpec((BB, V), lambda i: (i, 0)),
            pl.BlockSpec((E, V), lambda i: (0, 0)),
            pl.BlockSpec((1, E), lambda i: (0, 0)),
            pl.BlockSpec((BB, L), lambda i: (i, 0)),
            pl.BlockSpec((BB, L, E), lambda i: (i, 0, 0)),
        ],
        out_specs=pl.BlockSpec((BB, L), lambda i: (i, 0)),
        out_shape=jax.ShapeDtypeStruct((B, L), jnp.float32),
    )(message, W, b.reshape(1, E), mask, pooled3d)


def kernel(message, landmarks, mask, emb_table, W, b):
    B, L, T = landmarks.shape
    E = emb_table.shape[1]
    lm2d = landmarks.reshape(B, L * T)
    pooled = _sc_gather_pool(emb_table, lm2d, L, T)
    pooled3d = pooled.reshape(B, L, E)
    return _tc_score(message, W, b, mask, pooled3d)


# speedup vs baseline: 25.0822x; 1.2540x over previous
import functools

import jax
import jax.numpy as jnp
from jax import lax
from jax.experimental import pallas as pl
from jax.experimental.pallas import tpu as pltpu
from jax.experimental.pallas import tpu_sc as plsc

_LANES = 16
_NW = 32


def _sc_gather_pool(emb_table, lm2d, L, T):
    N, IDX = lm2d.shape
    E = emb_table.shape[1]
    SPW = N // _NW
    n_full, rem = divmod(IDX, 128)
    chunks = [(c * 128, 128) for c in range(n_full)]
    if rem:
        chunks.append((n_full * 128, rem))
    mesh = plsc.VectorSubcoreMesh(core_axis_name="c", subcore_axis_name="s")

    @functools.partial(
        pl.kernel,
        out_type=jax.ShapeDtypeStruct((N, L * E), jnp.float32),
        mesh=mesh,
        scratch_types=[
            pltpu.VMEM((SPW, IDX), jnp.int32),
            pltpu.VMEM((2, IDX, E), jnp.float32),
            pltpu.VMEM((2, L * E), jnp.float32),
            pltpu.SemaphoreType.DMA,
            pltpu.SemaphoreType.DMA,
            pltpu.SemaphoreType.DMA,
        ],
        compiler_params=pltpu.CompilerParams(use_tc_tiling_on_sc=False),
    )
    def k(table_hbm, lm_hbm, out_hbm, idx_v, rows_v, pout_v, g0, g1, osem):
        wid = lax.axis_index("s") * 2 + lax.axis_index("c")
        base = wid * SPW
        gsem = (g0, g1)

        def fire(s, b):
            for off, sz in chunks:
                pltpu.async_copy(
                    table_hbm.at[idx_v.at[s, pl.ds(off, sz)]],
                    rows_v.at[b, pl.ds(off, sz)],
                    gsem[b],
                )

        def drain_rows(b):
            pltpu.make_async_copy(
                table_hbm.at[pl.ds(0, IDX)], rows_v.at[b], gsem[b]).wait()

        def pool(s, b):
            def l_body(l, carry_l):
                r0 = l * T
                acc = [None] * 4
                for t in range(T):
                    h0 = rows_v[b, r0 + t, pl.ds(0, _LANES)]
                    h1 = rows_v[b, r0 + t, pl.ds(_LANES, _LANES)]
                    j = (t % 2) * 2
                    acc[j] = h0 if acc[j] is None else acc[j] + h0
                    acc[j + 1] = h1 if acc[j + 1] is None else acc[j + 1] + h1
                pout_v[b, pl.ds(l * E, _LANES)] = acc[0] + acc[2]
                pout_v[b, pl.ds(l * E + _LANES, _LANES)] = acc[1] + acc[3]
                return carry_l

            lax.fori_loop(0, L, l_body, 0)

        pltpu.sync_copy(lm_hbm.at[pl.ds(base, SPW)], idx_v)
        fire(0, 0)

        def step(i, carry):
            for b in range(2):
                s = 2 * i + b
                drain_rows(b)
                pl.when(s + 1 < SPW)(lambda: fire(s + 1, 1 - b))
                pl.when(s >= 2)(lambda: pltpu.make_async_copy(
                    pout_v.at[b], out_hbm.at[base + s - 2], osem).wait())
                pool(s, b)
                pltpu.async_copy(pout_v.at[b], out_hbm.at[base + s], osem)
            return carry

        lax.fori_loop(0, SPW // 2, step, 0)
        for b in range(2):
            pltpu.make_async_copy(
                pout_v.at[b], out_hbm.at[base + SPW - 2 + b], osem).wait()

    return k(emb_table, lm2d)


def _tc_score(message, W, b, mask, pooled3d):
    B, V = message.shape
    E = W.shape[0]
    L = mask.shape[1]
    BB = 256

    def body(msg_ref, w_ref, b_ref, mask_ref, pooled_ref, out_ref):
        m = lax.dot_general(
            msg_ref[...], w_ref[...], (((1,), (1,)), ((), ())),
            preferred_element_type=jnp.float32)
        m = jnp.maximum(m + b_ref[...], 0.0)
        p = jnp.maximum(pooled_ref[...], 0.0)
        s = jnp.sum(p * m[:, None, :], axis=2)
        s = s + (1.0 - mask_ref[...]) * (-1e36)
        mx = jnp.max(s, axis=1, keepdims=True)
        e = jnp.exp(s - mx)
        out_ref[...] = e / jnp.sum(e, axis=1, keepdims=True)

    return pl.pallas_call(
        body,
        grid=(B // BB,),
        in_specs=[
            pl.BlockSpec((BB, V), lambda i: (i, 0)),
            pl.BlockSpec((E, V), lambda i: (0, 0)),
            pl.BlockSpec((1, E), lambda i: (0, 0)),
            pl.BlockSpec((BB, L), lambda i: (i, 0)),
            pl.BlockSpec((BB, L, E), lambda i: (i, 0, 0)),
        ],
        out_specs=pl.BlockSpec((BB, L), lambda i: (i, 0)),
        out_shape=jax.ShapeDtypeStruct((B, L), jnp.float32),
    )(message, W, b.reshape(1, E), mask, pooled3d)


def kernel(message, landmarks, mask, emb_table, W, b):
    B, L, T = landmarks.shape
    E = emb_table.shape[1]
    lm2d = landmarks.reshape(B, L * T)
    pooled = _sc_gather_pool(emb_table, lm2d, L, T)
    pooled3d = pooled.reshape(B, L, E)
    return _tc_score(message, W, b, mask, pooled3d)


# speedup vs baseline: 34.7373x; 1.3849x over previous
import functools

import jax
import jax.numpy as jnp
from jax import lax
from jax.experimental import pallas as pl
from jax.experimental.pallas import tpu as pltpu
from jax.experimental.pallas import tpu_sc as plsc

_LANES = 16
_NW = 32


def _sc_score(emb_table, lm2d, msgr, L, T):
    N, IDX = lm2d.shape
    E = emb_table.shape[1]
    SPW = N // _NW
    n_full, rem = divmod(IDX, 128)
    chunks = [(c * 128, 128) for c in range(n_full)]
    if rem:
        chunks.append((n_full * 128, rem))
    mesh = plsc.VectorSubcoreMesh(core_axis_name="c", subcore_axis_name="s")

    LP = 64

    @functools.partial(
        pl.kernel,
        out_type=jax.ShapeDtypeStruct((N, LP), jnp.float32),
        mesh=mesh,
        scratch_types=[
            pltpu.VMEM((SPW, IDX), jnp.int32),
            pltpu.VMEM((2, IDX, E), jnp.float32),
            pltpu.VMEM((SPW, E), jnp.float32),
            pltpu.VMEM((_LANES * LP,), jnp.float32),
            pltpu.VMEM((SPW, LP), jnp.float32),
            pltpu.SemaphoreType.DMA,
            pltpu.SemaphoreType.DMA,
        ],
        compiler_params=pltpu.CompilerParams(use_tc_tiling_on_sc=False, needs_layout_passes=False),
    )
    def k(table_hbm, lm_hbm, msg_hbm, out_hbm, idx_v, rows_v, msg_v, pstage_v,
          sout_v, g0, g1):
        wid = lax.axis_index("s") * 2 + lax.axis_index("c")
        base = wid * SPW
        gsem = (g0, g1)
        col_idx = lax.broadcasted_iota(jnp.int32, (_LANES,), 0) * LP

        def fire(s, b):
            for off, sz in chunks:
                pltpu.async_copy(
                    table_hbm.at[idx_v.at[s, pl.ds(off, sz)]],
                    rows_v.at[b, pl.ds(off, sz)],
                    gsem[b],
                )

        def drain_rows(b):
            pltpu.make_async_copy(
                table_hbm.at[pl.ds(0, IDX)], rows_v.at[b], gsem[b]).wait()

        def pool_one(b, l, m0, m1):
            r0 = l * T
            acc = [None] * 4
            for t in range(T):
                h0 = rows_v[b, r0 + t, pl.ds(0, _LANES)]
                h1 = rows_v[b, r0 + t, pl.ds(_LANES, _LANES)]
                j = (t % 2) * 2
                acc[j] = h0 if acc[j] is None else acc[j] + h0
                acc[j + 1] = h1 if acc[j + 1] is None else acc[j + 1] + h1
            a0 = jnp.maximum(acc[0] + acc[2], 0.0)
            a1 = jnp.maximum(acc[1] + acc[3], 0.0)
            v = a0 * m0 + a1 * m1
            plsc.store_scatter(pstage_v, [col_idx + l], v)

        def score(s, b):
            m0 = msg_v[s, pl.ds(0, _LANES)]
            m1 = msg_v[s, pl.ds(_LANES, _LANES)]

            def l_body(i, carry_l):
                pool_one(b, 2 * i, m0, m1)
                pool_one(b, 2 * i + 1, m0, m1)
                return carry_l

            lax.fori_loop(0, L // 2, l_body, 0)
            for g in range(LP // _LANES):
                tot = None
                for e in range(_LANES):
                    c = pstage_v[pl.ds(e * LP + g * _LANES, _LANES)]
                    tot = c if tot is None else tot + c
                sout_v[s, pl.ds(g * _LANES, _LANES)] = tot

        pltpu.sync_copy(lm_hbm.at[pl.ds(base, SPW)], idx_v)
        pltpu.sync_copy(msg_hbm.at[pl.ds(base, SPW)], msg_v)
        fire(0, 0)

        def step(i, carry):
            for b in range(2):
                s = 2 * i + b
                drain_rows(b)
                pl.when(s + 1 < SPW)(lambda: fire(s + 1, 1 - b))
                score(s, b)
            return carry

        lax.fori_loop(0, SPW // 2, step, 0)
        pltpu.sync_copy(sout_v, out_hbm.at[pl.ds(base, SPW)])

    return k(emb_table, lm2d, msgr)


def _tc_linear(message, W, b):
    B, V = message.shape
    E = W.shape[0]
    BB = 256

    def body(msg_ref, w_ref, b_ref, out_ref):
        m = lax.dot_general(
            msg_ref[...], w_ref[...], (((1,), (1,)), ((), ())),
            preferred_element_type=jnp.float32)
        out_ref[...] = jnp.maximum(m + b_ref[...], 0.0)

    return pl.pallas_call(
        body,
        grid=(B // BB,),
        in_specs=[
            pl.BlockSpec((BB, V), lambda i: (i, 0)),
            pl.BlockSpec((E, V), lambda i: (0, 0)),
            pl.BlockSpec((1, E), lambda i: (0, 0)),
        ],
        out_specs=pl.BlockSpec((BB, E), lambda i: (i, 0)),
        out_shape=jax.ShapeDtypeStruct((B, E), jnp.float32),
    )(message, W, b.reshape(1, E))


def _tc_softmax(score_pad, mask):
    B, L = mask.shape

    def body(s_ref, mask_ref, out_ref):
        s = s_ref[:, :L] + (1.0 - mask_ref[...]) * (-1e36)
        mx = jnp.max(s, axis=1, keepdims=True)
        e = jnp.exp(s - mx)
        out_ref[...] = e / jnp.sum(e, axis=1, keepdims=True)

    return pl.pallas_call(
        body,
        out_shape=jax.ShapeDtypeStruct((B, L), jnp.float32),
    )(score_pad, mask)


def kernel(message, landmarks, mask, emb_table, W, b):
    B, L, T = landmarks.shape
    lm2d = landmarks.reshape(B, L * T)
    msgr = _tc_linear(message, W, b)
    score = _sc_score(emb_table, lm2d, msgr, L, T)
    return _tc_softmax(score, mask)
